# baseline (speedup 1.0000x reference)
import functools

import jax
import jax.numpy as jnp
from jax import lax
from jax.experimental import pallas as pl
from jax.experimental.pallas import tpu as pltpu
from jax.experimental.pallas import tpu_sc as plsc

_VOCAB = 1000000
_EMBED = 64
_B = 4096
_QLEN = 20
_QPAD = 24
_TLEN = 200
_IDXW = _QPAD + _TLEN
_NW = 32
_BPW = _B // _NW


def _sc_pool(idx_hbm, w_hbm, qout_hbm, tout_hbm, idx_v, rows_v, qout_v,
             tout_v, sem):
  wid = lax.axis_index("s") * 2 + lax.axis_index("c")
  base = wid * _BPW
  pltpu.sync_copy(idx_hbm.at[pl.ds(base, _BPW)], idx_v)

  def item(b, _):
    c1 = pltpu.async_copy(w_hbm.at[idx_v.at[b, pl.ds(0, _QPAD)]],
                          rows_v.at[pl.ds(0, _QPAD)], sem)
    c2 = pltpu.async_copy(w_hbm.at[idx_v.at[b, pl.ds(_QPAD, 128)]],
                          rows_v.at[pl.ds(_QPAD, 128)], sem)
    c3 = pltpu.async_copy(w_hbm.at[idx_v.at[b, pl.ds(_QPAD + 128, 72)]],
                          rows_v.at[pl.ds(_QPAD + 128, 72)], sem)
    c1.wait()
    c2.wait()
    c3.wait()
    for g in range(4):
      acc = rows_v[0, pl.ds(g * 16, 16)]
      for r in range(1, _QPAD):
        acc = acc + rows_v[r, pl.ds(g * 16, 16)]
      qout_v[b, pl.ds(g * 16, 16)] = acc

    def tstep(i, accs):
      r = _QPAD + i * 4
      out = []
      for g in range(4):
        a = accs[g]
        for rr in range(4):
          a = a + rows_v[r + rr, pl.ds(g * 16, 16)]
        out.append(a)
      return tuple(out)

    zero = jnp.zeros((16,), jnp.float32)
    taccs = lax.fori_loop(0, _TLEN // 4, tstep, (zero, zero, zero, zero))
    for g in range(4):
      tout_v[b, pl.ds(g * 16, 16)] = taccs[g]
    return 0

  lax.fori_loop(0, _BPW, item, 0)
  pltpu.sync_copy(qout_v, qout_hbm.at[pl.ds(base, _BPW)])
  pltpu.sync_copy(tout_v, tout_hbm.at[pl.ds(base, _BPW)])


_sc_pool_call = functools.partial(
    pl.kernel,
    out_type=[
        jax.ShapeDtypeStruct((_B, _EMBED), jnp.float32),
        jax.ShapeDtypeStruct((_B, _EMBED), jnp.float32),
    ],
    mesh=plsc.VectorSubcoreMesh(core_axis_name="c", subcore_axis_name="s"),
    compiler_params=pltpu.CompilerParams(use_tc_tiling_on_sc=False),
    scratch_types=[
        pltpu.VMEM((_BPW, _IDXW), jnp.int32),
        pltpu.VMEM((_IDXW, _EMBED), jnp.float32),
        pltpu.VMEM((_BPW, _EMBED), jnp.float32),
        pltpu.VMEM((_BPW, _EMBED), jnp.float32),
        pltpu.SemaphoreType.DMA,
    ],
)(_sc_pool)


_WBLK = 8192


def _wsum_body(w_ref, o_ref):
  i = pl.program_id(0)
  blk = w_ref[...]
  rid = lax.broadcasted_iota(jnp.int32, blk.shape, 0) + i * _WBLK
  blk = jnp.where(rid < _VOCAB + 1, blk, 0.0)

  @pl.when(i == 0)
  def _():
    o_ref[...] = jnp.zeros_like(o_ref)

  o_ref[...] += blk.sum(axis=0, keepdims=True)


def _wsum(W):
  grid = (_VOCAB + 1 + _WBLK - 1) // _WBLK
  return pl.pallas_call(
      _wsum_body,
      out_shape=jax.ShapeDtypeStruct((1, _EMBED), jnp.float32),
      grid=(grid,),
      in_specs=[pl.BlockSpec((_WBLK, _EMBED), lambda i: (i, 0))],
      out_specs=pl.BlockSpec((1, _EMBED), lambda i: (0, 0)),
  )(W)


def _combine_body(qraw_ref, traw_ref, wsum_ref, w0_ref, qlen_ref, tlen_ref,
                  s_ref, sneg_ref):
  w0 = w0_ref[...]
  wsum = wsum_ref[...] - w0
  qlen = qlen_ref[...].astype(jnp.float32)
  tlen = tlen_ref[...].astype(jnp.float32)
  q = (qraw_ref[...] - 4.0 * w0) / qlen
  t = traw_ref[...]
  s_ref[...] = jnp.sum((t / tlen) * q, axis=1, keepdims=True)
  sneg_ref[...] = jnp.sum((wsum - t) * q, axis=1, keepdims=True) * (
      1.0 / float(_VOCAB))


def _combine(qraw, traw, wsum, w0, qlen, tlen):
  return pl.pallas_call(
      _combine_body,
      out_shape=[
          jax.ShapeDtypeStruct((_B, 1), jnp.float32),
          jax.ShapeDtypeStruct((_B, 1), jnp.float32),
      ],
  )(qraw, traw, wsum, w0, qlen, tlen)


def kernel(q, q_len, text, text_len, W):
  idx = jnp.concatenate(
      [q, jnp.zeros((_B, _QPAD - _QLEN), jnp.int32), text], axis=1)
  qraw, traw = _sc_pool_call(idx, W)
  wsum = _wsum(W)
  w0 = lax.slice(W, (0, 0), (1, _EMBED))
  s, s_neg = _combine(qraw, traw, wsum, w0, q_len.reshape(_B, 1),
                      text_len.reshape(_B, 1))
  return (s.reshape(-1), s_neg.reshape(-1))


# speedup vs baseline: 1.1378x; 1.1378x over previous
import functools

import jax
import jax.numpy as jnp
from jax import lax
from jax.experimental import pallas as pl
from jax.experimental.pallas import tpu as pltpu
from jax.experimental.pallas import tpu_sc as plsc

_VOCAB = 1000000
_EMBED = 64
_B = 4096
_QLEN = 20
_QPAD = 24
_TLEN = 200
_IDXW = _QPAD + _TLEN
_NW = 32
_BPW = _B // _NW
_CHUNKS = ((0, _QPAD), (_QPAD, 128), (_QPAD + 128, 72))
_VPAD = 1000008


_TBLK = 4096


def _tr_body(wt_ref, w128_ref, wsum_ref):
  i = pl.program_id(0)
  blk = wt_ref[...]
  eye = (lax.broadcasted_iota(jnp.int32, (_EMBED, _EMBED), 0) ==
         lax.broadcasted_iota(jnp.int32, (_EMBED, _EMBED), 1)
         ).astype(jnp.float32)
  tr = lax.dot_general(blk, eye, (((0,), (0,)), ((), ())),
                       preferred_element_type=jnp.float32)
  w128_ref[:, 0:_EMBED] = tr
  cid = lax.broadcasted_iota(jnp.int32, blk.shape, 1) + i * _TBLK
  psum = jnp.sum(jnp.where(cid < _VOCAB + 1, blk, 0.0), axis=1,
                 keepdims=True)

  @pl.when(i == 0)
  def _():
    wsum_ref[...] = jnp.zeros_like(wsum_ref)

  wsum_ref[...] += psum


def _transpose_wsum(WT):
  grid = (_VPAD + _TBLK - 1) // _TBLK
  return pl.pallas_call(
      _tr_body,
      out_shape=[
          jax.ShapeDtypeStruct((_VPAD, 128), jnp.float32),
          jax.ShapeDtypeStruct((_EMBED, 1), jnp.float32),
      ],
      grid=(grid,),
      in_specs=[pl.BlockSpec((_EMBED, _TBLK), lambda i: (0, i))],
      out_specs=[
          pl.BlockSpec((_TBLK, 128), lambda i: (i, 0)),
          pl.BlockSpec((_EMBED, 1), lambda i: (0, 0)),
      ],
  )(WT)


def _sc_pool(idx_hbm, w_hbm, qout_hbm, tout_hbm, idx_v, rows0_v, rows1_v,
             qout_v, tout_v, sem0, sem1):
  wid = lax.axis_index("s") * 2 + lax.axis_index("c")
  base = pl.multiple_of(wid * (_BPW * _IDXW), 8)
  pltpu.sync_copy(idx_hbm.at[pl.ds(base, _BPW * _IDXW)], idx_v)
  obase = pl.multiple_of(wid * (_BPW * _EMBED), 8)

  bufs = (rows0_v, rows1_v)
  sems = (sem0, sem1)

  def issue(b, buf, sem):
    off = pl.multiple_of(b * _IDXW, 8)
    for (c0, cn) in _CHUNKS:
      pltpu.async_copy(w_hbm.at[idx_v.at[pl.ds(off + c0, cn)]],
                       buf.at[pl.ds(c0, cn)], sem)

  def drain(buf, sem):
    pltpu.make_async_copy(w_hbm.at[pl.ds(0, _IDXW)], buf, sem).wait()

  def pool(b, buf):
    oof = pl.multiple_of(b * _EMBED, 8)
    for g in range(4):
      acc = buf[0, pl.ds(g * 16, 16)]
      for r in range(1, _QPAD):
        acc = acc + buf[r, pl.ds(g * 16, 16)]
      qout_v[pl.ds(oof + g * 16, 16)] = acc
    for g in range(4):
      acc = buf[_QPAD, pl.ds(g * 16, 16)]
      for r in range(_QPAD + 1, _IDXW):
        acc = acc + buf[r, pl.ds(g * 16, 16)]
      tout_v[pl.ds(oof + g * 16, 16)] = acc

  issue(0, bufs[0], sems[0])
  issue(1, bufs[1], sems[1])

  def pair(g, _):
    for u in range(2):
      b = 2 * g + u
      drain(bufs[u], sems[u])
      pool(b, bufs[u])

      @pl.when(g < _BPW // 2 - 1)
      def _():
        issue(b + 2, bufs[u], sems[u])

    return 0

  lax.fori_loop(0, _BPW // 2, pair, 0)
  pltpu.sync_copy(qout_v, qout_hbm.at[pl.ds(obase, _BPW * _EMBED)])
  pltpu.sync_copy(tout_v, tout_hbm.at[pl.ds(obase, _BPW * _EMBED)])


_sc_pool_call = functools.partial(
    pl.kernel,
    out_type=[
        jax.ShapeDtypeStruct((_B * _EMBED,), jnp.float32),
        jax.ShapeDtypeStruct((_B * _EMBED,), jnp.float32),
    ],
    mesh=plsc.VectorSubcoreMesh(core_axis_name="c", subcore_axis_name="s"),
    scratch_types=[
        pltpu.VMEM((_BPW * _IDXW,), jnp.int32),
        pltpu.VMEM((_IDXW, 128), jnp.float32),
        pltpu.VMEM((_IDXW, 128), jnp.float32),
        pltpu.VMEM((_BPW * _EMBED,), jnp.float32),
        pltpu.VMEM((_BPW * _EMBED,), jnp.float32),
        pltpu.SemaphoreType.DMA,
        pltpu.SemaphoreType.DMA,
    ],
)(_sc_pool)


def _combine_body(qraw_ref, traw_ref, wsum_ref, w0_ref, qlen_ref, tlen_ref,
                  s_ref, sneg_ref):
  w0 = w0_ref[...]
  wsum = wsum_ref[...] - w0
  qlen = qlen_ref[...].astype(jnp.float32)
  tlen = tlen_ref[...].astype(jnp.float32)
  q = (qraw_ref[...] - 4.0 * w0) / qlen
  t = traw_ref[...]
  s_ref[...] = jnp.sum((t / tlen) * q, axis=1, keepdims=True)
  sneg_ref[...] = jnp.sum((wsum - t) * q, axis=1, keepdims=True) * (
      1.0 / float(_VOCAB))


def _combine(qraw, traw, wsum, w0, qlen, tlen):
  return pl.pallas_call(
      _combine_body,
      out_shape=[
          jax.ShapeDtypeStruct((_B, 1), jnp.float32),
          jax.ShapeDtypeStruct((_B, 1), jnp.float32),
      ],
  )(qraw, traw, wsum, w0, qlen, tlen)


def kernel(q, q_len, text, text_len, W):
  idx = jnp.concatenate(
      [q, jnp.zeros((_B, _QPAD - _QLEN), jnp.int32), text],
      axis=1).reshape(-1)
  w128, wsum = _transpose_wsum(W.T)
  qraw, traw = _sc_pool_call(idx, w128)
  w0 = lax.slice(W, (0, 0), (1, _EMBED))
  s, s_neg = _combine(qraw.reshape(_B, _EMBED), traw.reshape(_B, _EMBED),
                      wsum.reshape(1, _EMBED), w0, q_len.reshape(_B, 1),
                      text_len.reshape(_B, 1))
  return (s.reshape(-1), s_neg.reshape(-1))
